# baseline (speedup 1.0000x reference)
import functools

import numpy as np
import jax
import jax.numpy as jnp
from jax import lax
from jax.experimental import pallas as pl
from jax.experimental.pallas import tpu as pltpu
from jax.experimental.pallas import tpu_sc as plsc

_ANCHOR_SCALE = (8.0, 16.0, 32.0)
_ANCHOR_RATIO = (0.5, 1.0, 2.0)
_FEAT_STRIDE = 16
_RPN_MIN_SIZE = 16.0
_PRE_NMS_N = 6000
_POST_NMS_N = 300
_NMS_THRESH = 0.7
_IMG_W = 800.0
_IMG_H = 800.0
_MAP = 50
_N = _MAP * _MAP * 9
_NP = 22528
_NW = 16
_CH = _NP // _NW
_NVR = _CH // 16
_OUTN = 8192
_DUMP = 6144
_C_ROWS = 48
_C_COLS = 128


def _np_anchors():
    base_size = 16.0
    base = np.array([0.0, 0.0, base_size - 1.0, base_size - 1.0], dtype=np.float32)
    w = base[2] - base[0] + 1.0
    h = base[3] - base[1] + 1.0
    x_ctr = base[0] + 0.5 * (w - 1.0)
    y_ctr = base[1] + 0.5 * (h - 1.0)
    size = w * h
    anchors = []
    for r in _ANCHOR_RATIO:
        size_r = size / r
        ws = np.round(np.sqrt(size_r))
        hs = np.round(ws * r)
        for s in _ANCHOR_SCALE:
            wss = ws * s
            hss = hs * s
            anchors.append([x_ctr - 0.5 * (wss - 1.0), y_ctr - 0.5 * (hss - 1.0),
                            x_ctr + 0.5 * (wss - 1.0), y_ctr + 0.5 * (hss - 1.0)])
    anchors = np.array(anchors, dtype=np.float32)
    sx = np.arange(_MAP, dtype=np.float32) * _FEAT_STRIDE
    sy = np.arange(_MAP, dtype=np.float32) * _FEAT_STRIDE
    sxg, syg = np.meshgrid(sx, sy)
    shifts = np.stack([sxg.ravel(), syg.ravel(), sxg.ravel(), syg.ravel()], axis=1)
    all_anchors = (anchors[None, :, :] + shifts[:, None, :]).reshape(-1, 4)
    return all_anchors.astype(np.float32)


def _pad1d(col, fill):
    out = np.full((_NP,), fill, dtype=np.float32)
    out[:_N] = col
    return out


@functools.lru_cache(maxsize=1)
def _anchor_consts_1d():
    a = _np_anchors()
    widths = a[:, 2] - a[:, 0] + 1.0
    heights = a[:, 3] - a[:, 1] + 1.0
    ctr_x = a[:, 0] + 0.5 * widths
    ctr_y = a[:, 1] + 0.5 * heights
    return (_pad1d(widths, 16.0), _pad1d(heights, 16.0),
            _pad1d(ctr_x, 0.0), _pad1d(ctr_y, 0.0))


def _sc_sortable(f):
    m = lax.bitcast_convert_type(f, jnp.int32)
    neg = jnp.bitwise_xor(jnp.bitwise_not(m), jnp.int32(-2**31))
    return jnp.where(m >= 0, m, neg)


def _dgather16(v, idx):
    return lax.gather(
        v, idx.reshape(16, 1),
        dimension_numbers=lax.GatherDimensionNumbers(
            offset_dims=(), collapsed_slice_dims=(0,), start_index_map=(0,)),
        slice_sizes=(1,),
        mode=lax.GatherScatterMode.PROMISE_IN_BOUNDS)


def _sc_body(dxh, dyh, dwh, dhh, sh, wah, hah, cxh, cyh,
             odx, ody, odw, odh, osc, owa, oha, ocx, ocy, oidx,
             vdx, vdy, vdw, vdh, vs, vwa, vha, vcx, vcy,
             vv, vlin, vpos, cntbuf, posbuf, zbuf, shared):
    sid = lax.axis_index("s")
    base = sid * _CH
    iota16 = lax.iota(jnp.int32, 16)
    K = jnp.int32(_PRE_NMS_N)

    def butterfly(v):
        for k in (1, 2, 4, 8):
            v = v + _dgather16(v, jnp.bitwise_xor(iota16, jnp.int32(k)))
        return v

    def prefix16(v):
        for k in (1, 2, 4, 8):
            sh_ = _dgather16(v, jnp.maximum(iota16 - k, 0))
            v = v + jnp.where(iota16 >= k, sh_, 0)
        return v

    def b2i(m):
        return jnp.where(m, jnp.int32(1), jnp.int32(0))

    pltpu.sync_copy(dxh.at[pl.ds(base, _CH)], vdx)
    pltpu.sync_copy(dyh.at[pl.ds(base, _CH)], vdy)
    pltpu.sync_copy(dwh.at[pl.ds(base, _CH)], vdw)
    pltpu.sync_copy(dhh.at[pl.ds(base, _CH)], vdh)
    pltpu.sync_copy(sh.at[pl.ds(base, _CH)], vs)
    pltpu.sync_copy(wah.at[pl.ds(base, _CH)], vwa)
    pltpu.sync_copy(hah.at[pl.ds(base, _CH)], vha)
    pltpu.sync_copy(cxh.at[pl.ds(base, _CH)], vcx)
    pltpu.sync_copy(cyh.at[pl.ds(base, _CH)], vcy)

    def zfill(j, carry):
        zbuf[pl.ds(j * 16, 16)] = jnp.zeros((16,), jnp.int32)
        return carry
    lax.fori_loop(0, 49, zfill, jnp.int32(0))

    @pl.when(sid == 0)
    def _():
        pltpu.sync_copy(zbuf, shared)

    def transform(j, carry):
        sl = pl.ds(j * 16, 16)
        dx = vdx[sl]
        dy = vdy[sl]
        dw = vdw[sl]
        dh = vdh[sl]
        wa = vwa[sl]
        ha = vha[sl]
        cxa = vcx[sl]
        cya = vcy[sl]
        s_in = vs[sl]
        lin = base + j * 16 + iota16
        pred_ctr_x = dx * wa + cxa
        pred_ctr_y = dy * ha + cya
        pred_w = jnp.exp(dw) * wa
        pred_h = jnp.exp(dh) * ha
        x1 = jnp.clip(pred_ctr_x - 0.5 * pred_w, 0.0, _IMG_W - 1.0)
        y1 = jnp.clip(pred_ctr_y - 0.5 * pred_h, 0.0, _IMG_H - 1.0)
        x2 = jnp.clip(pred_ctr_x + 0.5 * pred_w, 0.0, _IMG_W - 1.0)
        y2 = jnp.clip(pred_ctr_y + 0.5 * pred_h, 0.0, _IMG_H - 1.0)
        size_ok = ((x2 - x1 + 1.0 >= _RPN_MIN_SIZE)
                   & (y2 - y1 + 1.0 >= _RPN_MIN_SIZE))
        s_eff = jnp.where(size_ok, s_in, jnp.float32(-1e9))
        s_eff = jnp.where(lin < _N, s_eff, jnp.float32(-jnp.inf))
        vv[sl] = _sc_sortable(s_eff + 0.0)
        vlin[sl] = lin
        return carry
    lax.fori_loop(0, _NVR, transform, jnp.int32(0))

    plsc.subcore_barrier()

    def global_count(slot, lane_cnt):
        cntbuf[...] = lane_cnt
        posbuf[...] = slot * 16 + iota16
        pltpu.sync_copy(cntbuf, shared.at[posbuf], add=True)
        plsc.subcore_barrier()
        off = pl.multiple_of(slot * 16, 16)
        pltpu.sync_copy(shared.at[pl.ds(off, 16)], cntbuf)
        return butterfly(cntbuf[...])

    def lanes_gt(t):
        def body(j, acc):
            return acc + b2i(vv[pl.ds(j * 16, 16)] > t)
        return lax.fori_loop(0, _NVR, body, jnp.zeros((16,), jnp.int32))

    int_min = jnp.full((16,), -2**31, jnp.int32)
    tot_m1 = global_count(jnp.int32(0), lanes_gt(jnp.full((16,), -1, jnp.int32)))
    l0 = jnp.where(tot_m1 >= K, jnp.int32(-1), int_min)

    def vstep(t, l):
        cand = l + (jnp.int32(1) << (jnp.int32(30) - t))
        tot = global_count(t + 1, lanes_gt(cand))
        return jnp.where(tot >= K, cand, l)
    lfin = lax.fori_loop(0, 31, vstep, l0)
    a_thr = lfin + jnp.int32(1)
    n_gt = global_count(jnp.int32(32), lanes_gt(a_thr))
    need = K - n_gt

    def lanes_eq_lt(cb):
        def body(j, acc):
            sl = pl.ds(j * 16, 16)
            m = (vv[sl] == a_thr) & (vlin[sl] < cb)
            return acc + b2i(m)
        return lax.fori_loop(0, _NVR, body, jnp.zeros((16,), jnp.int32))

    def istep(t, l):
        cand = l + (jnp.int32(1) << (jnp.int32(14) - t))
        tot = global_count(t + 33, lanes_eq_lt(cand))
        return jnp.where(tot < need, cand, l)
    cfin = lax.fori_loop(0, 15, istep, jnp.zeros((16,), jnp.int32)) + jnp.int32(1)

    def lanes_member(j):
        sl = pl.ds(j * 16, 16)
        v = vv[sl]
        return (v > a_thr) | ((v == a_thr) & (vlin[sl] < cfin))

    def mcount(j, acc):
        return acc + b2i(lanes_member(j))
    mvec = lax.fori_loop(0, _NVR, mcount, jnp.zeros((16,), jnp.int32))
    m_w = butterfly(mvec)
    my_slot = jnp.where(iota16 == sid, m_w, 0)
    mall = global_count(jnp.int32(48), my_slot)
    del mall
    off_w = butterfly(jnp.where(iota16 < sid, cntbuf[...], 0))

    def posloop(j, running):
        sl = pl.ds(j * 16, 16)
        member = lanes_member(j)
        mi = b2i(member)
        incl = prefix16(mi)
        lin = vlin[sl]
        dump = _DUMP + jnp.bitwise_and(lin, jnp.int32(2047))
        pos = jnp.where(member, off_w + running + incl - 1, dump)
        r2 = j // 8
        c2 = (j % 8) * 16
        vpos[r2, pl.ds(c2, 16)] = pos
        return running + butterfly(mi)
    lax.fori_loop(0, _NVR, posloop, jnp.zeros((16,), jnp.int32))

    def scat(dst, src, j, c):
        pltpu.sync_copy(src.at[pl.ds(c * 128, 128)], dst.at[vpos.at[j]])

    def scatloop(j, carry):
        c = j
        scat(odx, vdx, j, c)
        scat(ody, vdy, j, c)
        scat(odw, vdw, j, c)
        scat(odh, vdh, j, c)
        scat(osc, vs, j, c)
        scat(owa, vwa, j, c)
        scat(oha, vha, j, c)
        scat(ocx, vcx, j, c)
        scat(ocy, vcy, j, c)
        scat(oidx, vlin, j, c)
        return carry
    lax.fori_loop(0, _CH // 128, scatloop, jnp.int32(0))


def _sc_stage(dx, dy, dw, dh, s, wa, ha, cx, cy):
    mesh = plsc.VectorSubcoreMesh(core_axis_name="c", subcore_axis_name="s",
                                  num_cores=1, num_subcores=_NW)
    f32 = jnp.float32
    out_type = [jax.ShapeDtypeStruct((_OUTN,), f32) for _ in range(9)]
    out_type.append(jax.ShapeDtypeStruct((_OUTN,), jnp.int32))
    vm_f = lambda: pltpu.VMEM((_CH,), f32)
    vm_i = lambda: pltpu.VMEM((_CH,), jnp.int32)
    scratch = [vm_f() for _ in range(9)] + [vm_i(), vm_i(),
               pltpu.VMEM((_CH // 128, 128), jnp.int32),
               pltpu.VMEM((16,), jnp.int32), pltpu.VMEM((16,), jnp.int32),
               pltpu.VMEM((784,), jnp.int32),
               pltpu.VMEM_SHARED((784,), jnp.int32)]
    fn = pl.kernel(_sc_body, out_type=tuple(out_type), mesh=mesh,
                   scratch_types=tuple(scratch))
    return fn(dx, dy, dw, dh, s, wa, ha, cx, cy)


def _tc_body(dx_ref, dy_ref, dw_ref, dh_ref, s_ref, wa_ref, ha_ref, cx_ref,
             cy_ref, idx_ref, ob_ref, os_ref, valid_ref):
    dx = dx_ref[...]
    dy = dy_ref[...]
    dw = dw_ref[...]
    dh = dh_ref[...]
    s_in = s_ref[...]
    wa = wa_ref[...]
    ha = ha_ref[...]
    cxa = cx_ref[...]
    cya = cy_ref[...]
    oidx = idx_ref[...]

    lin = (lax.broadcasted_iota(jnp.int32, (_C_ROWS, _C_COLS), 0) * _C_COLS
           + lax.broadcasted_iota(jnp.int32, (_C_ROWS, _C_COLS), 1))
    real = lin < _PRE_NMS_N
    oidx_eff = jnp.where(real, oidx, -1)

    pred_ctr_x = dx * wa + cxa
    pred_ctr_y = dy * ha + cya
    pred_w = jnp.exp(dw) * wa
    pred_h = jnp.exp(dh) * ha
    x1 = jnp.clip(pred_ctr_x - 0.5 * pred_w, 0.0, _IMG_W - 1.0)
    y1 = jnp.clip(pred_ctr_y - 0.5 * pred_h, 0.0, _IMG_H - 1.0)
    x2 = jnp.clip(pred_ctr_x + 0.5 * pred_w, 0.0, _IMG_W - 1.0)
    y2 = jnp.clip(pred_ctr_y + 0.5 * pred_h, 0.0, _IMG_H - 1.0)
    ws = x2 - x1 + 1.0
    hs = y2 - y1 + 1.0
    size_ok = (ws >= _RPN_MIN_SIZE) & (hs >= _RPN_MIN_SIZE)
    s_eff = jnp.where(size_ok, s_in, jnp.float32(-1e9))
    s_eff = jnp.where(real, s_eff, -jnp.inf)
    areas = (x2 - x1) * (y2 - y1)

    neg_inf = jnp.float32(-jnp.inf)
    big_i = jnp.int32(2**30)

    def select(valid):
        masked = jnp.where(valid, s_eff, neg_inf)
        m = jnp.max(masked)
        idx = jnp.min(jnp.where(masked == m, oidx_eff, big_i))
        return m, idx

    def emit_and_suppress(t, i, s_out, valid):
        onehot = oidx_eff == i
        x1i = jnp.sum(jnp.where(onehot, x1, 0.0))
        y1i = jnp.sum(jnp.where(onehot, y1, 0.0))
        x2i = jnp.sum(jnp.where(onehot, x2, 0.0))
        y2i = jnp.sum(jnp.where(onehot, y2, 0.0))
        ai = jnp.sum(jnp.where(onehot, areas, 0.0))
        ob_ref[t, 0] = x1i
        ob_ref[t, 1] = y1i
        ob_ref[t, 2] = x2i
        ob_ref[t, 3] = y2i
        os_ref[t] = s_out
        xx1 = jnp.maximum(x1i, x1)
        yy1 = jnp.maximum(y1i, y1)
        xx2 = jnp.minimum(x2i, x2)
        yy2 = jnp.minimum(y2i, y2)
        iw = jnp.maximum(xx2 - xx1, 0.0)
        ih = jnp.maximum(yy2 - yy1, 0.0)
        inter = iw * ih
        iou = inter / (ai + areas - inter + jnp.float32(1e-8))
        return valid & (iou <= _NMS_THRESH) & jnp.logical_not(onehot)

    m0, i0 = select(real)
    valid_ref[...] = emit_and_suppress(0, i0, m0, real).astype(jnp.float32)

    def nms_step(t, carry):
        valid = valid_ref[...] != 0.0
        m, idx = select(valid)
        empty = m == neg_inf
        i = jnp.where(empty, i0, idx)
        s_out = jnp.where(empty, m0, m)
        valid_ref[...] = emit_and_suppress(t, i, s_out, valid).astype(jnp.float32)
        return carry

    lax.fori_loop(1, _POST_NMS_N, nms_step, jnp.int32(0))


def _tc_stage(arrs):
    return pl.pallas_call(
        _tc_body,
        out_shape=(jax.ShapeDtypeStruct((_POST_NMS_N, 4), jnp.float32),
                   jax.ShapeDtypeStruct((_POST_NMS_N,), jnp.float32)),
        out_specs=(pl.BlockSpec(memory_space=pltpu.SMEM),
                   pl.BlockSpec(memory_space=pltpu.SMEM)),
        scratch_shapes=[pltpu.VMEM((_C_ROWS, _C_COLS), jnp.float32)],
    )(*arrs)


def kernel(delta, score):
    wa, ha, cxa, cya = _anchor_consts_1d()
    d = jnp.reshape(delta, (-1, 4))
    padz = jnp.zeros((_NP - _N,), jnp.float32)
    cat = lambda c: jnp.concatenate([c, padz])
    dx = cat(d[:, 0])
    dy = cat(d[:, 1])
    dw = cat(d[:, 2])
    dh = cat(d[:, 3])
    s = jnp.concatenate([jnp.reshape(score[:, :, :, 9:], (-1,)),
                         jnp.full((_NP - _N,), -jnp.inf)])
    outs = _sc_stage(dx, dy, dw, dh, s,
                     jnp.asarray(wa), jnp.asarray(ha),
                     jnp.asarray(cxa), jnp.asarray(cya))
    comp = [jnp.reshape(o[:_C_ROWS * _C_COLS], (_C_ROWS, _C_COLS))
            for o in outs]
    return _tc_stage(comp)


# speedup vs baseline: 1.0159x; 1.0159x over previous
import functools

import numpy as np
import jax
import jax.numpy as jnp
from jax import lax
from jax.experimental import pallas as pl
from jax.experimental.pallas import tpu as pltpu
from jax.experimental.pallas import tpu_sc as plsc

_ANCHOR_SCALE = (8.0, 16.0, 32.0)
_ANCHOR_RATIO = (0.5, 1.0, 2.0)
_FEAT_STRIDE = 16
_RPN_MIN_SIZE = 16.0
_PRE_NMS_N = 6000
_POST_NMS_N = 300
_NMS_THRESH = 0.7
_IMG_W = 800.0
_IMG_H = 800.0
_MAP = 50
_N = _MAP * _MAP * 9
_NP = 22528
_NW = 16
_CH = _NP // _NW
_NVR = _CH // 16
_OUTN = 8192
_DUMP = 6144
_C_ROWS = 48
_C_COLS = 128


def _np_anchors():
    base_size = 16.0
    base = np.array([0.0, 0.0, base_size - 1.0, base_size - 1.0], dtype=np.float32)
    w = base[2] - base[0] + 1.0
    h = base[3] - base[1] + 1.0
    x_ctr = base[0] + 0.5 * (w - 1.0)
    y_ctr = base[1] + 0.5 * (h - 1.0)
    size = w * h
    anchors = []
    for r in _ANCHOR_RATIO:
        size_r = size / r
        ws = np.round(np.sqrt(size_r))
        hs = np.round(ws * r)
        for s in _ANCHOR_SCALE:
            wss = ws * s
            hss = hs * s
            anchors.append([x_ctr - 0.5 * (wss - 1.0), y_ctr - 0.5 * (hss - 1.0),
                            x_ctr + 0.5 * (wss - 1.0), y_ctr + 0.5 * (hss - 1.0)])
    anchors = np.array(anchors, dtype=np.float32)
    sx = np.arange(_MAP, dtype=np.float32) * _FEAT_STRIDE
    sy = np.arange(_MAP, dtype=np.float32) * _FEAT_STRIDE
    sxg, syg = np.meshgrid(sx, sy)
    shifts = np.stack([sxg.ravel(), syg.ravel(), sxg.ravel(), syg.ravel()], axis=1)
    all_anchors = (anchors[None, :, :] + shifts[:, None, :]).reshape(-1, 4)
    return all_anchors.astype(np.float32)


def _pad1d(col, fill):
    out = np.full((_NP,), fill, dtype=np.float32)
    out[:_N] = col
    return out


@functools.lru_cache(maxsize=1)
def _anchor_consts_1d():
    a = _np_anchors()
    widths = a[:, 2] - a[:, 0] + 1.0
    heights = a[:, 3] - a[:, 1] + 1.0
    ctr_x = a[:, 0] + 0.5 * widths
    ctr_y = a[:, 1] + 0.5 * heights
    return (_pad1d(widths, 16.0), _pad1d(heights, 16.0),
            _pad1d(ctr_x, 0.0), _pad1d(ctr_y, 0.0))


def _sc_sortable(f):
    m = lax.bitcast_convert_type(f, jnp.int32)
    neg = jnp.bitwise_xor(jnp.bitwise_not(m), jnp.int32(-2**31))
    return jnp.where(m >= 0, m, neg)


def _dgather16(v, idx):
    return lax.gather(
        v, idx.reshape(16, 1),
        dimension_numbers=lax.GatherDimensionNumbers(
            offset_dims=(), collapsed_slice_dims=(0,), start_index_map=(0,)),
        slice_sizes=(1,),
        mode=lax.GatherScatterMode.PROMISE_IN_BOUNDS)


def _sc_body(dxh, dyh, dwh, dhh, sh, wah, hah, cxh, cyh,
             odx, ody, odw, odh, osc, owa, oha, ocx, ocy, oidx,
             vdx, vdy, vdw, vdh, vs, vwa, vha, vcx, vcy,
             vv, vlin, vpos, cntbuf, posbuf, zbuf, shared, sem):
    sid = lax.axis_index("s")
    base = sid * _CH
    iota16 = lax.iota(jnp.int32, 16)
    K = jnp.int32(_PRE_NMS_N)

    def butterfly(v):
        for k in (1, 2, 4, 8):
            v = v + _dgather16(v, jnp.bitwise_xor(iota16, jnp.int32(k)))
        return v

    def prefix16(v):
        for k in (1, 2, 4, 8):
            sh_ = _dgather16(v, jnp.maximum(iota16 - k, 0))
            v = v + jnp.where(iota16 >= k, sh_, 0)
        return v

    def b2i(m):
        return jnp.where(m, jnp.int32(1), jnp.int32(0))

    pltpu.sync_copy(dxh.at[pl.ds(base, _CH)], vdx)
    pltpu.sync_copy(dyh.at[pl.ds(base, _CH)], vdy)
    pltpu.sync_copy(dwh.at[pl.ds(base, _CH)], vdw)
    pltpu.sync_copy(dhh.at[pl.ds(base, _CH)], vdh)
    pltpu.sync_copy(sh.at[pl.ds(base, _CH)], vs)
    pltpu.sync_copy(wah.at[pl.ds(base, _CH)], vwa)
    pltpu.sync_copy(hah.at[pl.ds(base, _CH)], vha)
    pltpu.sync_copy(cxh.at[pl.ds(base, _CH)], vcx)
    pltpu.sync_copy(cyh.at[pl.ds(base, _CH)], vcy)

    def zfill(j, carry):
        zbuf[pl.ds(j * 16, 16)] = jnp.zeros((16,), jnp.int32)
        return carry
    lax.fori_loop(0, 49, zfill, jnp.int32(0))

    @pl.when(sid == 0)
    def _():
        pltpu.sync_copy(zbuf, shared)

    def transform(j, carry):
        sl = pl.ds(j * 16, 16)
        dx = vdx[sl]
        dy = vdy[sl]
        dw = vdw[sl]
        dh = vdh[sl]
        wa = vwa[sl]
        ha = vha[sl]
        cxa = vcx[sl]
        cya = vcy[sl]
        s_in = vs[sl]
        lin = base + j * 16 + iota16
        pred_ctr_x = dx * wa + cxa
        pred_ctr_y = dy * ha + cya
        pred_w = jnp.exp(dw) * wa
        pred_h = jnp.exp(dh) * ha
        x1 = jnp.clip(pred_ctr_x - 0.5 * pred_w, 0.0, _IMG_W - 1.0)
        y1 = jnp.clip(pred_ctr_y - 0.5 * pred_h, 0.0, _IMG_H - 1.0)
        x2 = jnp.clip(pred_ctr_x + 0.5 * pred_w, 0.0, _IMG_W - 1.0)
        y2 = jnp.clip(pred_ctr_y + 0.5 * pred_h, 0.0, _IMG_H - 1.0)
        size_ok = ((x2 - x1 + 1.0 >= _RPN_MIN_SIZE)
                   & (y2 - y1 + 1.0 >= _RPN_MIN_SIZE))
        s_eff = jnp.where(size_ok, s_in, jnp.float32(-1e9))
        s_eff = jnp.where(lin < _N, s_eff, jnp.float32(-jnp.inf))
        vv[sl] = _sc_sortable(s_eff + 0.0)
        vlin[sl] = lin
        return carry
    lax.fori_loop(0, _NVR, transform, jnp.int32(0), unroll=4)

    plsc.subcore_barrier()

    def global_count(slot, lane_cnt):
        cntbuf[...] = lane_cnt
        posbuf[...] = slot * 16 + iota16
        pltpu.sync_copy(cntbuf, shared.at[posbuf], add=True)
        plsc.subcore_barrier()
        off = pl.multiple_of(slot * 16, 16)
        pltpu.sync_copy(shared.at[pl.ds(off, 16)], cntbuf)
        return butterfly(cntbuf[...])

    def lanes_gt(t):
        def body(j, acc):
            return acc + b2i(vv[pl.ds(j * 16, 16)] > t)
        return lax.fori_loop(0, _NVR, body, jnp.zeros((16,), jnp.int32), unroll=8)

    int_min = jnp.full((16,), -2**31, jnp.int32)
    tot_m1 = global_count(jnp.int32(0), lanes_gt(jnp.full((16,), -1, jnp.int32)))
    l0 = jnp.where(tot_m1 >= K, jnp.int32(-1), int_min)

    def vstep(t, l):
        cand = l + (jnp.int32(1) << (jnp.int32(30) - t))
        tot = global_count(t + 1, lanes_gt(cand))
        return jnp.where(tot >= K, cand, l)
    lfin = lax.fori_loop(0, 31, vstep, l0)
    a_thr = lfin + jnp.int32(1)
    n_gt = global_count(jnp.int32(32), lanes_gt(a_thr))
    need = K - n_gt

    def lanes_eq_lt(cb):
        def body(j, acc):
            sl = pl.ds(j * 16, 16)
            m = (vv[sl] == a_thr) & (vlin[sl] < cb)
            return acc + b2i(m)
        return lax.fori_loop(0, _NVR, body, jnp.zeros((16,), jnp.int32), unroll=8)

    def istep(t, l):
        cand = l + (jnp.int32(1) << (jnp.int32(14) - t))
        tot = global_count(t + 33, lanes_eq_lt(cand))
        return jnp.where(tot < need, cand, l)
    cfin = lax.fori_loop(0, 15, istep, jnp.zeros((16,), jnp.int32)) + jnp.int32(1)

    def lanes_member(j):
        sl = pl.ds(j * 16, 16)
        v = vv[sl]
        return (v > a_thr) | ((v == a_thr) & (vlin[sl] < cfin))

    def mcount(j, acc):
        return acc + b2i(lanes_member(j))
    mvec = lax.fori_loop(0, _NVR, mcount, jnp.zeros((16,), jnp.int32), unroll=8)
    m_w = butterfly(mvec)
    my_slot = jnp.where(iota16 == sid, m_w, 0)
    mall = global_count(jnp.int32(48), my_slot)
    del mall
    off_w = butterfly(jnp.where(iota16 < sid, cntbuf[...], 0))

    def posloop(j, running):
        sl = pl.ds(j * 16, 16)
        member = lanes_member(j)
        mi = b2i(member)
        incl = prefix16(mi)
        lin = vlin[sl]
        dump = _DUMP + jnp.bitwise_and(lin, jnp.int32(2047))
        pos = jnp.where(member, off_w + running + incl - 1, dump)
        r2 = j // 8
        c2 = (j % 8) * 16
        vpos[r2, pl.ds(c2, 16)] = pos
        return running + butterfly(mi)
    lax.fori_loop(0, _NVR, posloop, jnp.zeros((16,), jnp.int32), unroll=4)

    def scatloop(j, carry):
        sl = pl.ds(j * 128, 128)
        row = vpos.at[j]
        cps = [pltpu.async_copy(src.at[sl], dst.at[row], sem)
               for src, dst in ((vdx, odx), (vdy, ody), (vdw, odw),
                                (vdh, odh), (vs, osc), (vwa, owa),
                                (vha, oha), (vcx, ocx), (vcy, ocy),
                                (vlin, oidx))]
        for cp in cps:
            cp.wait()
        return carry
    lax.fori_loop(0, _CH // 128, scatloop, jnp.int32(0))


def _sc_stage(dx, dy, dw, dh, s, wa, ha, cx, cy):
    mesh = plsc.VectorSubcoreMesh(core_axis_name="c", subcore_axis_name="s",
                                  num_cores=1, num_subcores=_NW)
    f32 = jnp.float32
    out_type = [jax.ShapeDtypeStruct((_OUTN,), f32) for _ in range(9)]
    out_type.append(jax.ShapeDtypeStruct((_OUTN,), jnp.int32))
    vm_f = lambda: pltpu.VMEM((_CH,), f32)
    vm_i = lambda: pltpu.VMEM((_CH,), jnp.int32)
    scratch = [vm_f() for _ in range(9)] + [vm_i(), vm_i(),
               pltpu.VMEM((_CH // 128, 128), jnp.int32),
               pltpu.VMEM((16,), jnp.int32), pltpu.VMEM((16,), jnp.int32),
               pltpu.VMEM((784,), jnp.int32),
               pltpu.VMEM_SHARED((784,), jnp.int32),
               pltpu.SemaphoreType.DMA]
    fn = pl.kernel(_sc_body, out_type=tuple(out_type), mesh=mesh,
                   scratch_types=tuple(scratch))
    return fn(dx, dy, dw, dh, s, wa, ha, cx, cy)


def _tc_body(dx_ref, dy_ref, dw_ref, dh_ref, s_ref, wa_ref, ha_ref, cx_ref,
             cy_ref, idx_ref, ob_ref, os_ref, valid_ref,
             x1_ref, y1_ref, x2_ref, y2_ref, ar_ref):
    dx = dx_ref[...]
    dy = dy_ref[...]
    dw = dw_ref[...]
    dh = dh_ref[...]
    s_in = s_ref[...]
    wa = wa_ref[...]
    ha = ha_ref[...]
    cxa = cx_ref[...]
    cya = cy_ref[...]

    lin = (lax.broadcasted_iota(jnp.int32, (_C_ROWS, _C_COLS), 0) * _C_COLS
           + lax.broadcasted_iota(jnp.int32, (_C_ROWS, _C_COLS), 1))
    col = lax.broadcasted_iota(jnp.int32, (1, _C_COLS), 1)
    real = lin < _PRE_NMS_N

    pred_ctr_x = dx * wa + cxa
    pred_ctr_y = dy * ha + cya
    pred_w = jnp.exp(dw) * wa
    pred_h = jnp.exp(dh) * ha
    x1 = jnp.clip(pred_ctr_x - 0.5 * pred_w, 0.0, _IMG_W - 1.0)
    y1 = jnp.clip(pred_ctr_y - 0.5 * pred_h, 0.0, _IMG_H - 1.0)
    x2 = jnp.clip(pred_ctr_x + 0.5 * pred_w, 0.0, _IMG_W - 1.0)
    y2 = jnp.clip(pred_ctr_y + 0.5 * pred_h, 0.0, _IMG_H - 1.0)
    ws = x2 - x1 + 1.0
    hs = y2 - y1 + 1.0
    size_ok = (ws >= _RPN_MIN_SIZE) & (hs >= _RPN_MIN_SIZE)
    s_eff = jnp.where(size_ok, s_in, jnp.float32(-1e9))
    s_eff = jnp.where(real, s_eff, -jnp.inf)
    areas = (x2 - x1) * (y2 - y1)
    x1_ref[...] = x1
    y1_ref[...] = y1
    x2_ref[...] = x2
    y2_ref[...] = y2
    ar_ref[...] = areas

    neg_inf = jnp.float32(-jnp.inf)
    big_i = jnp.int32(2**30)

    def select(valid):
        masked = jnp.where(valid, s_eff, neg_inf)
        m = jnp.max(masked)
        p = jnp.min(jnp.where(masked == m, lin, big_i))
        return m, p

    def extract(ref, r, c):
        row = ref[pl.ds(r, 1), :]
        return jnp.sum(jnp.where(col == c, row, 0.0))

    def emit_and_suppress(t, p, s_out, valid):
        r = p // _C_COLS
        c = p % _C_COLS
        x1i = extract(x1_ref, r, c)
        y1i = extract(y1_ref, r, c)
        x2i = extract(x2_ref, r, c)
        y2i = extract(y2_ref, r, c)
        ai = extract(ar_ref, r, c)
        ob_ref[t, 0] = x1i
        ob_ref[t, 1] = y1i
        ob_ref[t, 2] = x2i
        ob_ref[t, 3] = y2i
        os_ref[t] = s_out
        xx1 = jnp.maximum(x1i, x1)
        yy1 = jnp.maximum(y1i, y1)
        xx2 = jnp.minimum(x2i, x2)
        yy2 = jnp.minimum(y2i, y2)
        iw = jnp.maximum(xx2 - xx1, 0.0)
        ih = jnp.maximum(yy2 - yy1, 0.0)
        inter = iw * ih
        iou = inter / (ai + areas - inter + jnp.float32(1e-8))
        return valid & (iou <= _NMS_THRESH) & (lin != p)

    m0, p0 = select(real)
    valid_ref[...] = emit_and_suppress(0, p0, m0, real).astype(jnp.float32)

    def nms_step(t, carry):
        valid = valid_ref[...] != 0.0
        m, p = select(valid)
        empty = m == neg_inf
        p = jnp.where(empty, p0, p)
        s_out = jnp.where(empty, m0, m)
        valid_ref[...] = emit_and_suppress(t, p, s_out, valid).astype(jnp.float32)
        return carry

    lax.fori_loop(1, _POST_NMS_N, nms_step, jnp.int32(0))


def _tc_stage(arrs):
    vm = lambda: pltpu.VMEM((_C_ROWS, _C_COLS), jnp.float32)
    return pl.pallas_call(
        _tc_body,
        out_shape=(jax.ShapeDtypeStruct((_POST_NMS_N, 4), jnp.float32),
                   jax.ShapeDtypeStruct((_POST_NMS_N,), jnp.float32)),
        out_specs=(pl.BlockSpec(memory_space=pltpu.SMEM),
                   pl.BlockSpec(memory_space=pltpu.SMEM)),
        scratch_shapes=[vm(), vm(), vm(), vm(), vm(), vm()],
    )(*arrs)


def kernel(delta, score):
    wa, ha, cxa, cya = _anchor_consts_1d()
    d = jnp.reshape(delta, (-1, 4))
    padz = jnp.zeros((_NP - _N,), jnp.float32)
    cat = lambda c: jnp.concatenate([c, padz])
    dx = cat(d[:, 0])
    dy = cat(d[:, 1])
    dw = cat(d[:, 2])
    dh = cat(d[:, 3])
    s = jnp.concatenate([jnp.reshape(score[:, :, :, 9:], (-1,)),
                         jnp.full((_NP - _N,), -jnp.inf)])
    outs = _sc_stage(dx, dy, dw, dh, s,
                     jnp.asarray(wa), jnp.asarray(ha),
                     jnp.asarray(cxa), jnp.asarray(cya))
    comp = [jnp.reshape(o[:_C_ROWS * _C_COLS], (_C_ROWS, _C_COLS))
            for o in outs]
    return _tc_stage(comp)


# speedup vs baseline: 1.0497x; 1.0332x over previous
import functools

import numpy as np
import jax
import jax.numpy as jnp
from jax import lax
from jax.experimental import pallas as pl
from jax.experimental.pallas import tpu as pltpu
from jax.experimental.pallas import tpu_sc as plsc

_ANCHOR_SCALE = (8.0, 16.0, 32.0)
_ANCHOR_RATIO = (0.5, 1.0, 2.0)
_FEAT_STRIDE = 16
_RPN_MIN_SIZE = 16.0
_PRE_NMS_N = 6000
_POST_NMS_N = 300
_NMS_THRESH = 0.7
_IMG_W = 800.0
_IMG_H = 800.0
_MAP = 50
_N = _MAP * _MAP * 9
_NP = 22528
_NW = 16
_CH = _NP // _NW
_NVR = _CH // 16
_OUTN = 8192
_DUMP = 6144
_C_ROWS = 48
_C_COLS = 128


def _np_anchors():
    base_size = 16.0
    base = np.array([0.0, 0.0, base_size - 1.0, base_size - 1.0], dtype=np.float32)
    w = base[2] - base[0] + 1.0
    h = base[3] - base[1] + 1.0
    x_ctr = base[0] + 0.5 * (w - 1.0)
    y_ctr = base[1] + 0.5 * (h - 1.0)
    size = w * h
    anchors = []
    for r in _ANCHOR_RATIO:
        size_r = size / r
        ws = np.round(np.sqrt(size_r))
        hs = np.round(ws * r)
        for s in _ANCHOR_SCALE:
            wss = ws * s
            hss = hs * s
            anchors.append([x_ctr - 0.5 * (wss - 1.0), y_ctr - 0.5 * (hss - 1.0),
                            x_ctr + 0.5 * (wss - 1.0), y_ctr + 0.5 * (hss - 1.0)])
    anchors = np.array(anchors, dtype=np.float32)
    sx = np.arange(_MAP, dtype=np.float32) * _FEAT_STRIDE
    sy = np.arange(_MAP, dtype=np.float32) * _FEAT_STRIDE
    sxg, syg = np.meshgrid(sx, sy)
    shifts = np.stack([sxg.ravel(), syg.ravel(), sxg.ravel(), syg.ravel()], axis=1)
    all_anchors = (anchors[None, :, :] + shifts[:, None, :]).reshape(-1, 4)
    return all_anchors.astype(np.float32)


def _pad1d(col, fill):
    out = np.full((_NP,), fill, dtype=np.float32)
    out[:_N] = col
    return out


@functools.lru_cache(maxsize=1)
def _anchor_consts_1d():
    a = _np_anchors()
    widths = a[:, 2] - a[:, 0] + 1.0
    heights = a[:, 3] - a[:, 1] + 1.0
    ctr_x = a[:, 0] + 0.5 * widths
    ctr_y = a[:, 1] + 0.5 * heights
    return (_pad1d(widths, 16.0), _pad1d(heights, 16.0),
            _pad1d(ctr_x, 0.0), _pad1d(ctr_y, 0.0))


def _sc_sortable(f):
    m = lax.bitcast_convert_type(f, jnp.int32)
    neg = jnp.bitwise_xor(jnp.bitwise_not(m), jnp.int32(-2**31))
    return jnp.where(m >= 0, m, neg)


def _dgather16(v, idx):
    return lax.gather(
        v, idx.reshape(16, 1),
        dimension_numbers=lax.GatherDimensionNumbers(
            offset_dims=(), collapsed_slice_dims=(0,), start_index_map=(0,)),
        slice_sizes=(1,),
        mode=lax.GatherScatterMode.PROMISE_IN_BOUNDS)


def _sc_body(dxh, dyh, dwh, dhh, sh, wah, hah, cxh, cyh,
             odx, ody, odw, odh, osc, owa, oha, ocx, ocy,
             vdx, vdy, vdw, vdh, vs, vwa, vha, vcx, vcy,
             vv, vlin, vpos, cnt3buf, rdbuf, shared, sem):
    sid = lax.axis_index("s")
    base = sid * _CH
    iota16 = lax.iota(jnp.int32, 16)
    K = jnp.int32(_PRE_NMS_N)

    def butterfly(v):
        for k in (1, 2, 4, 8):
            v = v + _dgather16(v, jnp.bitwise_xor(iota16, jnp.int32(k)))
        return v

    def prefix16(v):
        for k in (1, 2, 4, 8):
            sh_ = _dgather16(v, jnp.maximum(iota16 - k, 0))
            v = v + jnp.where(iota16 >= k, sh_, 0)
        return v

    def b2i(m):
        return jnp.where(m, jnp.int32(1), jnp.int32(0))

    in_cps = [pltpu.async_copy(href.at[pl.ds(base, _CH)], vref, sem)
              for href, vref in ((dxh, vdx), (dyh, vdy), (dwh, vdw),
                                 (dhh, vdh), (sh, vs), (wah, vwa),
                                 (hah, vha), (cxh, vcx), (cyh, vcy))]
    for cp in in_cps:
        cp.wait()

    def transform(j, carry):
        sl = pl.ds(j * 16, 16)
        dx = vdx[sl]
        dy = vdy[sl]
        dw = vdw[sl]
        dh = vdh[sl]
        wa = vwa[sl]
        ha = vha[sl]
        cxa = vcx[sl]
        cya = vcy[sl]
        s_in = vs[sl]
        lin = base + j * 16 + iota16
        pred_ctr_x = dx * wa + cxa
        pred_ctr_y = dy * ha + cya
        pred_w = jnp.exp(dw) * wa
        pred_h = jnp.exp(dh) * ha
        x1 = jnp.clip(pred_ctr_x - 0.5 * pred_w, 0.0, _IMG_W - 1.0)
        y1 = jnp.clip(pred_ctr_y - 0.5 * pred_h, 0.0, _IMG_H - 1.0)
        x2 = jnp.clip(pred_ctr_x + 0.5 * pred_w, 0.0, _IMG_W - 1.0)
        y2 = jnp.clip(pred_ctr_y + 0.5 * pred_h, 0.0, _IMG_H - 1.0)
        size_ok = ((x2 - x1 + 1.0 >= _RPN_MIN_SIZE)
                   & (y2 - y1 + 1.0 >= _RPN_MIN_SIZE))
        s_eff = jnp.where(size_ok, s_in, jnp.float32(-1e9))
        s_eff = jnp.where(lin < _N, s_eff, jnp.float32(-jnp.inf))
        vv[sl] = _sc_sortable(s_eff + 0.0)
        vlin[sl] = lin
        return carry
    lax.fori_loop(0, _NVR, transform, jnp.int32(0), unroll=4)

    def global_count3(step, c0, c1, c2):
        cnt3buf[pl.ds(0, 16)] = c0
        cnt3buf[pl.ds(16, 16)] = c1
        cnt3buf[pl.ds(32, 16)] = c2
        woff = pl.multiple_of(step * 768 + sid * 48, 16)
        pltpu.sync_copy(cnt3buf, shared.at[pl.ds(woff, 48)])
        plsc.subcore_barrier()
        roff = pl.multiple_of(step * 768, 16)
        pltpu.sync_copy(shared.at[pl.ds(roff, 768)], rdbuf)

        def red(w, accs):
            a0, a1, a2 = accs
            return (a0 + rdbuf[pl.ds(w * 48, 16)],
                    a1 + rdbuf[pl.ds(w * 48 + 16, 16)],
                    a2 + rdbuf[pl.ds(w * 48 + 32, 16)])
        z = jnp.zeros((16,), jnp.int32)
        a0, a1, a2 = lax.fori_loop(0, _NW, red, (z, z, z), unroll=4)
        return butterfly(a0), butterfly(a1), butterfly(a2)

    def lanes_gt3(t0, t1, t2):
        def body(j, accs):
            a0, a1, a2 = accs
            v = vv[pl.ds(j * 16, 16)]
            return (a0 + b2i(v > t0), a1 + b2i(v > t1), a2 + b2i(v > t2))
        z = jnp.zeros((16,), jnp.int32)
        return lax.fori_loop(0, _NVR, body, (z, z, z), unroll=8)

    int_min = jnp.full((16,), -2**31, jnp.int32)
    neg1 = jnp.full((16,), -1, jnp.int32)
    c0, _, _ = lanes_gt3(neg1, neg1, neg1)
    tot_m1, _, _ = global_count3(jnp.int32(0), c0, c0, c0)
    l = jnp.where(tot_m1 >= K, neg1, int_min)

    def vstep(t, l):
        hi = jnp.int32(30) - 2 * t
        lo = hi - 1
        ca = l + (jnp.int32(1) << hi)
        cb = l + (jnp.int32(1) << lo)
        cc = ca + (jnp.int32(1) << lo)
        n_a, n_b, n_c = lanes_gt3(ca, cb, cc)
        ta, tb, tc = global_count3(t + 1, n_a, n_b, n_c)
        return jnp.where(tc >= K, cc,
                         jnp.where(ta >= K, ca,
                                   jnp.where(tb >= K, cb, l)))
    l = lax.fori_loop(0, 15, vstep, l)
    cz = l + jnp.int32(1)
    n_z0, _, _ = lanes_gt3(cz, cz, cz)
    tz, _, _ = global_count3(jnp.int32(16), n_z0, n_z0, n_z0)
    l = jnp.where(tz >= K, cz, l)
    a_thr = l + jnp.int32(1)

    def lanes_gt_eq(a):
        def body(j, accs):
            a0, a1 = accs
            v = vv[pl.ds(j * 16, 16)]
            return (a0 + b2i(v > a), a1 + b2i(v == a))
        z = jnp.zeros((16,), jnp.int32)
        return lax.fori_loop(0, _NVR, body, (z, z), unroll=8)
    g_c, e_c = lanes_gt_eq(a_thr)
    n_gt, _, _ = global_count3(jnp.int32(17), g_c, e_c, e_c)
    need = K - n_gt

    def lanes_eq_lt3(c0_, c1_, c2_):
        def body(j, accs):
            a0, a1, a2 = accs
            sl = pl.ds(j * 16, 16)
            eq = vv[sl] == a_thr
            lin = vlin[sl]
            return (a0 + b2i(eq & (lin < c0_)),
                    a1 + b2i(eq & (lin < c1_)),
                    a2 + b2i(eq & (lin < c2_)))
        z = jnp.zeros((16,), jnp.int32)
        return lax.fori_loop(0, _NVR, body, (z, z, z), unroll=8)

    def istep(t, li):
        hi = jnp.int32(14) - 2 * t
        lo = hi - 1
        ca = li + (jnp.int32(1) << hi)
        cb = li + (jnp.int32(1) << lo)
        cc = ca + (jnp.int32(1) << lo)
        n_a, n_b, n_c = lanes_eq_lt3(ca, cb, cc)
        ta, tb, tc = global_count3(t + 18, n_a, n_b, n_c)
        return jnp.where(tc < need, cc,
                         jnp.where(ta < need, ca,
                                   jnp.where(tb < need, cb, li)))
    li = lax.fori_loop(0, 7, istep, jnp.zeros((16,), jnp.int32))
    ciz = li + jnp.int32(1)
    n_i0, _, _ = lanes_eq_lt3(ciz, ciz, ciz)
    tiz, _, _ = global_count3(jnp.int32(25), n_i0, n_i0, n_i0)
    li = jnp.where(tiz < need, ciz, li)
    cfin = li + jnp.int32(1)

    def lanes_member(j):
        sl = pl.ds(j * 16, 16)
        v = vv[sl]
        return (v > a_thr) | ((v == a_thr) & (vlin[sl] < cfin))

    def mcount(j, acc):
        return acc + b2i(lanes_member(j))
    mvec = lax.fori_loop(0, _NVR, mcount, jnp.zeros((16,), jnp.int32),
                         unroll=8)
    cnt3buf[pl.ds(0, 16)] = mvec
    cnt3buf[pl.ds(16, 16)] = mvec
    cnt3buf[pl.ds(32, 16)] = mvec
    woff = pl.multiple_of(26 * 768 + sid * 48, 16)
    pltpu.sync_copy(cnt3buf, shared.at[pl.ds(woff, 48)])
    plsc.subcore_barrier()
    roff = pl.multiple_of(26 * 768, 16)
    pltpu.sync_copy(shared.at[pl.ds(roff, 768)], rdbuf)

    def offred(w, acc):
        row = rdbuf[pl.ds(w * 48, 16)]
        return acc + jnp.where(w < sid, row, 0)
    offacc = lax.fori_loop(0, _NW, offred, jnp.zeros((16,), jnp.int32),
                           unroll=4)
    off_w = butterfly(offacc)

    def posloop(j, running):
        sl = pl.ds(j * 16, 16)
        member = lanes_member(j)
        mi = b2i(member)
        incl = prefix16(mi)
        lin = vlin[sl]
        dump = _DUMP + jnp.bitwise_and(lin, jnp.int32(2047))
        pos = jnp.where(member, off_w + running + incl - 1, dump)
        r2 = j // 8
        c2 = (j % 8) * 16
        vpos[r2, pl.ds(c2, 16)] = pos
        return running + butterfly(mi)
    lax.fori_loop(0, _NVR, posloop, jnp.zeros((16,), jnp.int32), unroll=4)

    pairs = ((vdx, odx), (vdy, ody), (vdw, odw), (vdh, odh), (vs, osc),
             (vwa, owa), (vha, oha), (vcx, ocx), (vcy, ocy))
    chunks = _CH // 128
    group = 3
    for g in range(0, chunks, group):
        cps = []
        for j in range(g, min(g + group, chunks)):
            sl = pl.ds(j * 128, 128)
            row = vpos.at[j]
            cps.extend(pltpu.async_copy(src.at[sl], dst.at[row], sem)
                       for src, dst in pairs)
        for cp in cps:
            cp.wait()


def _sc_stage(dx, dy, dw, dh, s, wa, ha, cx, cy):
    mesh = plsc.VectorSubcoreMesh(core_axis_name="c", subcore_axis_name="s",
                                  num_cores=1, num_subcores=_NW)
    f32 = jnp.float32
    out_type = tuple(jax.ShapeDtypeStruct((_OUTN,), f32) for _ in range(9))
    vm_f = lambda: pltpu.VMEM((_CH,), f32)
    vm_i = lambda: pltpu.VMEM((_CH,), jnp.int32)
    scratch = [vm_f() for _ in range(9)] + [vm_i(), vm_i(),
               pltpu.VMEM((_CH // 128, 128), jnp.int32),
               pltpu.VMEM((48,), jnp.int32),
               pltpu.VMEM((768,), jnp.int32),
               pltpu.VMEM_SHARED((27 * 768,), jnp.int32),
               pltpu.SemaphoreType.DMA]
    fn = pl.kernel(_sc_body, out_type=out_type, mesh=mesh,
                   scratch_types=tuple(scratch))
    return fn(dx, dy, dw, dh, s, wa, ha, cx, cy)


def _tc_body(dx_ref, dy_ref, dw_ref, dh_ref, s_ref, wa_ref, ha_ref, cx_ref,
             cy_ref, ob_ref, os_ref, valid_ref,
             x1_ref, y1_ref, x2_ref, y2_ref, ar_ref):
    dx = dx_ref[...]
    dy = dy_ref[...]
    dw = dw_ref[...]
    dh = dh_ref[...]
    s_in = s_ref[...]
    wa = wa_ref[...]
    ha = ha_ref[...]
    cxa = cx_ref[...]
    cya = cy_ref[...]

    lin = (lax.broadcasted_iota(jnp.int32, (_C_ROWS, _C_COLS), 0) * _C_COLS
           + lax.broadcasted_iota(jnp.int32, (_C_ROWS, _C_COLS), 1))
    col = lax.broadcasted_iota(jnp.int32, (1, _C_COLS), 1)
    real = lin < _PRE_NMS_N

    pred_ctr_x = dx * wa + cxa
    pred_ctr_y = dy * ha + cya
    pred_w = jnp.exp(dw) * wa
    pred_h = jnp.exp(dh) * ha
    x1 = jnp.clip(pred_ctr_x - 0.5 * pred_w, 0.0, _IMG_W - 1.0)
    y1 = jnp.clip(pred_ctr_y - 0.5 * pred_h, 0.0, _IMG_H - 1.0)
    x2 = jnp.clip(pred_ctr_x + 0.5 * pred_w, 0.0, _IMG_W - 1.0)
    y2 = jnp.clip(pred_ctr_y + 0.5 * pred_h, 0.0, _IMG_H - 1.0)
    ws = x2 - x1 + 1.0
    hs = y2 - y1 + 1.0
    size_ok = (ws >= _RPN_MIN_SIZE) & (hs >= _RPN_MIN_SIZE)
    s_eff = jnp.where(size_ok, s_in, jnp.float32(-1e9))
    s_eff = jnp.where(real, s_eff, -jnp.inf)
    areas = (x2 - x1) * (y2 - y1)
    x1_ref[...] = x1
    y1_ref[...] = y1
    x2_ref[...] = x2
    y2_ref[...] = y2
    ar_ref[...] = areas

    neg_inf = jnp.float32(-jnp.inf)
    big_i = jnp.int32(2**30)

    def select(valid):
        masked = jnp.where(valid, s_eff, neg_inf)
        m = jnp.max(masked)
        p = jnp.min(jnp.where(masked == m, lin, big_i))
        return m, p

    def extract(ref, r, c):
        row = ref[pl.ds(r, 1), :]
        return jnp.sum(jnp.where(col == c, row, 0.0))

    def emit_and_suppress(t, p, s_out, valid):
        r = p // _C_COLS
        c = p % _C_COLS
        x1i = extract(x1_ref, r, c)
        y1i = extract(y1_ref, r, c)
        x2i = extract(x2_ref, r, c)
        y2i = extract(y2_ref, r, c)
        ai = extract(ar_ref, r, c)
        ob_ref[t, 0] = x1i
        ob_ref[t, 1] = y1i
        ob_ref[t, 2] = x2i
        ob_ref[t, 3] = y2i
        os_ref[t] = s_out
        xx1 = jnp.maximum(x1i, x1)
        yy1 = jnp.maximum(y1i, y1)
        xx2 = jnp.minimum(x2i, x2)
        yy2 = jnp.minimum(y2i, y2)
        iw = jnp.maximum(xx2 - xx1, 0.0)
        ih = jnp.maximum(yy2 - yy1, 0.0)
        inter = iw * ih
        iou = inter / (ai + areas - inter + jnp.float32(1e-8))
        return valid & (iou <= _NMS_THRESH) & (lin != p)

    m0, p0 = select(real)
    valid_ref[...] = emit_and_suppress(0, p0, m0, real).astype(jnp.float32)

    def nms_step(t, carry):
        valid = valid_ref[...] != 0.0
        m, p = select(valid)
        empty = m == neg_inf
        p = jnp.where(empty, p0, p)
        s_out = jnp.where(empty, m0, m)
        valid_ref[...] = emit_and_suppress(t, p, s_out, valid).astype(jnp.float32)
        return carry

    lax.fori_loop(1, _POST_NMS_N, nms_step, jnp.int32(0))


def _tc_stage(arrs):
    vm = lambda: pltpu.VMEM((_C_ROWS, _C_COLS), jnp.float32)
    return pl.pallas_call(
        _tc_body,
        out_shape=(jax.ShapeDtypeStruct((_POST_NMS_N, 4), jnp.float32),
                   jax.ShapeDtypeStruct((_POST_NMS_N,), jnp.float32)),
        out_specs=(pl.BlockSpec(memory_space=pltpu.SMEM),
                   pl.BlockSpec(memory_space=pltpu.SMEM)),
        scratch_shapes=[vm(), vm(), vm(), vm(), vm(), vm()],
    )(*arrs)


def kernel(delta, score):
    wa, ha, cxa, cya = _anchor_consts_1d()
    d = jnp.reshape(delta, (-1, 4))
    padz = jnp.zeros((_NP - _N,), jnp.float32)
    cat = lambda c: jnp.concatenate([c, padz])
    dx = cat(d[:, 0])
    dy = cat(d[:, 1])
    dw = cat(d[:, 2])
    dh = cat(d[:, 3])
    s = jnp.concatenate([jnp.reshape(score[:, :, :, 9:], (-1,)),
                         jnp.full((_NP - _N,), -jnp.inf)])
    outs = _sc_stage(dx, dy, dw, dh, s,
                     jnp.asarray(wa), jnp.asarray(ha),
                     jnp.asarray(cxa), jnp.asarray(cya))
    comp = [jnp.reshape(o[:_C_ROWS * _C_COLS], (_C_ROWS, _C_COLS))
            for o in outs]
    return _tc_stage(comp)


# speedup vs baseline: 4.4684x; 4.2570x over previous
import functools

import numpy as np
import jax
import jax.numpy as jnp
from jax import lax
from jax.experimental import pallas as pl
from jax.experimental.pallas import tpu as pltpu
from jax.experimental.pallas import tpu_sc as plsc

_ANCHOR_SCALE = (8.0, 16.0, 32.0)
_ANCHOR_RATIO = (0.5, 1.0, 2.0)
_FEAT_STRIDE = 16
_RPN_MIN_SIZE = 16.0
_PRE_NMS_N = 6000
_POST_NMS_N = 300
_NMS_THRESH = 0.7
_IMG_W = 800.0
_IMG_H = 800.0
_MAP = 50
_N = _MAP * _MAP * 9
_NP = 22528
_NW = 16
_CH = _NP // _NW
_NVR = _CH // 16
_OUTN = 8192
_DUMP = 6144
_C_ROWS = 48
_C_COLS = 128


def _np_anchors():
    base_size = 16.0
    base = np.array([0.0, 0.0, base_size - 1.0, base_size - 1.0], dtype=np.float32)
    w = base[2] - base[0] + 1.0
    h = base[3] - base[1] + 1.0
    x_ctr = base[0] + 0.5 * (w - 1.0)
    y_ctr = base[1] + 0.5 * (h - 1.0)
    size = w * h
    anchors = []
    for r in _ANCHOR_RATIO:
        size_r = size / r
        ws = np.round(np.sqrt(size_r))
        hs = np.round(ws * r)
        for s in _ANCHOR_SCALE:
            wss = ws * s
            hss = hs * s
            anchors.append([x_ctr - 0.5 * (wss - 1.0), y_ctr - 0.5 * (hss - 1.0),
                            x_ctr + 0.5 * (wss - 1.0), y_ctr + 0.5 * (hss - 1.0)])
    anchors = np.array(anchors, dtype=np.float32)
    sx = np.arange(_MAP, dtype=np.float32) * _FEAT_STRIDE
    sy = np.arange(_MAP, dtype=np.float32) * _FEAT_STRIDE
    sxg, syg = np.meshgrid(sx, sy)
    shifts = np.stack([sxg.ravel(), syg.ravel(), sxg.ravel(), syg.ravel()], axis=1)
    all_anchors = (anchors[None, :, :] + shifts[:, None, :]).reshape(-1, 4)
    return all_anchors.astype(np.float32)


def _pad1d(col, fill):
    out = np.full((_NP,), fill, dtype=np.float32)
    out[:_N] = col
    return out


@functools.lru_cache(maxsize=1)
def _anchor_consts_1d():
    a = _np_anchors()
    widths = a[:, 2] - a[:, 0] + 1.0
    heights = a[:, 3] - a[:, 1] + 1.0
    ctr_x = a[:, 0] + 0.5 * widths
    ctr_y = a[:, 1] + 0.5 * heights
    return (_pad1d(widths, 16.0), _pad1d(heights, 16.0),
            _pad1d(ctr_x, 0.0), _pad1d(ctr_y, 0.0))


def _sc_sortable(f):
    m = lax.bitcast_convert_type(f, jnp.int32)
    neg = jnp.bitwise_xor(jnp.bitwise_not(m), jnp.int32(-2**31))
    return jnp.where(m >= 0, m, neg)


def _dgather16(v, idx):
    return lax.gather(
        v, idx.reshape(16, 1),
        dimension_numbers=lax.GatherDimensionNumbers(
            offset_dims=(), collapsed_slice_dims=(0,), start_index_map=(0,)),
        slice_sizes=(1,),
        mode=lax.GatherScatterMode.PROMISE_IN_BOUNDS)


def _sc_body(dxh, dyh, dwh, dhh, sh, wah, hah, cxh, cyh,
             odx, ody, odw, odh, osc, owa, oha, ocx, ocy,
             vdx, vdy, vdw, vdh, vs, vwa, vha, vcx, vcy,
             vv, vlin, vpos, cnt3buf, rdbuf, shared,
             sh_dx, sh_dy, sh_dw, sh_dh, sh_s, sh_wa, sh_ha, sh_cx, sh_cy,
             sem):
    sid = lax.axis_index("s")
    base = sid * _CH
    iota16 = lax.iota(jnp.int32, 16)
    K = jnp.int32(_PRE_NMS_N)

    def butterfly(v):
        for k in (1, 2, 4, 8):
            v = v + _dgather16(v, jnp.bitwise_xor(iota16, jnp.int32(k)))
        return v

    def prefix16(v):
        for k in (1, 2, 4, 8):
            sh_ = _dgather16(v, jnp.maximum(iota16 - k, 0))
            v = v + jnp.where(iota16 >= k, sh_, 0)
        return v

    def b2i(m):
        return jnp.where(m, jnp.int32(1), jnp.int32(0))

    in_cps = [pltpu.async_copy(href.at[pl.ds(base, _CH)], vref, sem)
              for href, vref in ((dxh, vdx), (dyh, vdy), (dwh, vdw),
                                 (dhh, vdh), (sh, vs), (wah, vwa),
                                 (hah, vha), (cxh, vcx), (cyh, vcy))]
    for cp in in_cps:
        cp.wait()

    def transform(j, carry):
        sl = pl.ds(j * 16, 16)
        dx = vdx[sl]
        dy = vdy[sl]
        dw = vdw[sl]
        dh = vdh[sl]
        wa = vwa[sl]
        ha = vha[sl]
        cxa = vcx[sl]
        cya = vcy[sl]
        s_in = vs[sl]
        lin = base + j * 16 + iota16
        pred_ctr_x = dx * wa + cxa
        pred_ctr_y = dy * ha + cya
        pred_w = jnp.exp(dw) * wa
        pred_h = jnp.exp(dh) * ha
        x1 = jnp.clip(pred_ctr_x - 0.5 * pred_w, 0.0, _IMG_W - 1.0)
        y1 = jnp.clip(pred_ctr_y - 0.5 * pred_h, 0.0, _IMG_H - 1.0)
        x2 = jnp.clip(pred_ctr_x + 0.5 * pred_w, 0.0, _IMG_W - 1.0)
        y2 = jnp.clip(pred_ctr_y + 0.5 * pred_h, 0.0, _IMG_H - 1.0)
        size_ok = ((x2 - x1 + 1.0 >= _RPN_MIN_SIZE)
                   & (y2 - y1 + 1.0 >= _RPN_MIN_SIZE))
        s_eff = jnp.where(size_ok, s_in, jnp.float32(-1e9))
        s_eff = jnp.where(lin < _N, s_eff, jnp.float32(-jnp.inf))
        vv[sl] = _sc_sortable(s_eff + 0.0)
        vlin[sl] = lin
        return carry
    lax.fori_loop(0, _NVR, transform, jnp.int32(0), unroll=4)

    def global_count3(step, c0, c1, c2):
        cnt3buf[pl.ds(0, 16)] = c0
        cnt3buf[pl.ds(16, 16)] = c1
        cnt3buf[pl.ds(32, 16)] = c2
        woff = pl.multiple_of(step * 768 + sid * 48, 16)
        pltpu.sync_copy(cnt3buf, shared.at[pl.ds(woff, 48)])
        plsc.subcore_barrier()
        roff = pl.multiple_of(step * 768, 16)
        pltpu.sync_copy(shared.at[pl.ds(roff, 768)], rdbuf)

        def red(w, accs):
            a0, a1, a2 = accs
            return (a0 + rdbuf[pl.ds(w * 48, 16)],
                    a1 + rdbuf[pl.ds(w * 48 + 16, 16)],
                    a2 + rdbuf[pl.ds(w * 48 + 32, 16)])
        z = jnp.zeros((16,), jnp.int32)
        a0, a1, a2 = lax.fori_loop(0, _NW, red, (z, z, z), unroll=4)
        return butterfly(a0), butterfly(a1), butterfly(a2)

    def lanes_gt3(t0, t1, t2):
        def body(j, accs):
            a0, a1, a2 = accs
            v = vv[pl.ds(j * 16, 16)]
            return (a0 + b2i(v > t0), a1 + b2i(v > t1), a2 + b2i(v > t2))
        z = jnp.zeros((16,), jnp.int32)
        return lax.fori_loop(0, _NVR, body, (z, z, z), unroll=8)

    int_min = jnp.full((16,), -2**31, jnp.int32)
    neg1 = jnp.full((16,), -1, jnp.int32)
    c0, _, _ = lanes_gt3(neg1, neg1, neg1)
    tot_m1, _, _ = global_count3(jnp.int32(0), c0, c0, c0)
    l = jnp.where(tot_m1 >= K, neg1, int_min)

    def vstep(t, l):
        hi = jnp.int32(30) - 2 * t
        lo = hi - 1
        ca = l + (jnp.int32(1) << hi)
        cb = l + (jnp.int32(1) << lo)
        cc = ca + (jnp.int32(1) << lo)
        n_a, n_b, n_c = lanes_gt3(ca, cb, cc)
        ta, tb, tc = global_count3(t + 1, n_a, n_b, n_c)
        return jnp.where(tc >= K, cc,
                         jnp.where(ta >= K, ca,
                                   jnp.where(tb >= K, cb, l)))
    l = lax.fori_loop(0, 15, vstep, l)
    cz = l + jnp.int32(1)
    n_z0, _, _ = lanes_gt3(cz, cz, cz)
    tz, _, _ = global_count3(jnp.int32(16), n_z0, n_z0, n_z0)
    l = jnp.where(tz >= K, cz, l)
    a_thr = l + jnp.int32(1)

    def lanes_gt_eq(a):
        def body(j, accs):
            a0, a1 = accs
            v = vv[pl.ds(j * 16, 16)]
            return (a0 + b2i(v > a), a1 + b2i(v == a))
        z = jnp.zeros((16,), jnp.int32)
        return lax.fori_loop(0, _NVR, body, (z, z), unroll=8)
    g_c, e_c = lanes_gt_eq(a_thr)
    n_gt, _, _ = global_count3(jnp.int32(17), g_c, e_c, e_c)
    need = K - n_gt

    def lanes_eq_lt3(c0_, c1_, c2_):
        def body(j, accs):
            a0, a1, a2 = accs
            sl = pl.ds(j * 16, 16)
            eq = vv[sl] == a_thr
            lin = vlin[sl]
            return (a0 + b2i(eq & (lin < c0_)),
                    a1 + b2i(eq & (lin < c1_)),
                    a2 + b2i(eq & (lin < c2_)))
        z = jnp.zeros((16,), jnp.int32)
        return lax.fori_loop(0, _NVR, body, (z, z, z), unroll=8)

    def istep(t, li):
        hi = jnp.int32(14) - 2 * t
        lo = hi - 1
        ca = li + (jnp.int32(1) << hi)
        cb = li + (jnp.int32(1) << lo)
        cc = ca + (jnp.int32(1) << lo)
        n_a, n_b, n_c = lanes_eq_lt3(ca, cb, cc)
        ta, tb, tc = global_count3(t + 18, n_a, n_b, n_c)
        return jnp.where(tc < need, cc,
                         jnp.where(ta < need, ca,
                                   jnp.where(tb < need, cb, li)))
    li = lax.fori_loop(0, 7, istep, jnp.zeros((16,), jnp.int32))
    ciz = li + jnp.int32(1)
    n_i0, _, _ = lanes_eq_lt3(ciz, ciz, ciz)
    tiz, _, _ = global_count3(jnp.int32(25), n_i0, n_i0, n_i0)
    li = jnp.where(tiz < need, ciz, li)
    cfin = li + jnp.int32(1)

    def lanes_member(j):
        sl = pl.ds(j * 16, 16)
        v = vv[sl]
        return (v > a_thr) | ((v == a_thr) & (vlin[sl] < cfin))

    def mcount(j, acc):
        return acc + b2i(lanes_member(j))
    mvec = lax.fori_loop(0, _NVR, mcount, jnp.zeros((16,), jnp.int32),
                         unroll=8)
    cnt3buf[pl.ds(0, 16)] = mvec
    cnt3buf[pl.ds(16, 16)] = mvec
    cnt3buf[pl.ds(32, 16)] = mvec
    woff = pl.multiple_of(26 * 768 + sid * 48, 16)
    pltpu.sync_copy(cnt3buf, shared.at[pl.ds(woff, 48)])
    plsc.subcore_barrier()
    roff = pl.multiple_of(26 * 768, 16)
    pltpu.sync_copy(shared.at[pl.ds(roff, 768)], rdbuf)

    def offred(w, acc):
        row = rdbuf[pl.ds(w * 48, 16)]
        return acc + jnp.where(w < sid, row, 0)
    offacc = lax.fori_loop(0, _NW, offred, jnp.zeros((16,), jnp.int32),
                           unroll=4)
    off_w = butterfly(offacc)

    def posloop(j, running):
        sl = pl.ds(j * 16, 16)
        member = lanes_member(j)
        mi = b2i(member)
        incl = prefix16(mi)
        lin = vlin[sl]
        dump = _DUMP + jnp.bitwise_and(lin, jnp.int32(2047))
        pos = jnp.where(member, off_w + running + incl - 1, dump)
        r2 = j // 8
        c2 = (j % 8) * 16
        vpos[r2, pl.ds(c2, 16)] = pos
        return running + butterfly(mi)
    lax.fori_loop(0, _NVR, posloop, jnp.zeros((16,), jnp.int32), unroll=4)

    pairs = ((vdx, sh_dx), (vdy, sh_dy), (vdw, sh_dw), (vdh, sh_dh),
             (vs, sh_s), (vwa, sh_wa), (vha, sh_ha), (vcx, sh_cx),
             (vcy, sh_cy))
    chunks = _CH // 128
    for j in range(chunks):
        sl = pl.ds(j * 128, 128)
        row = vpos.at[j]
        cps = [pltpu.async_copy(src.at[sl], dst.at[row], sem)
               for src, dst in pairs]
        for cp in cps:
            cp.wait()
    plsc.subcore_barrier()
    wbase = pl.multiple_of(sid * 384, 16)
    stage = vdx.at[pl.ds(0, 384)]
    outs = ((sh_dx, odx), (sh_dy, ody), (sh_dw, odw), (sh_dh, odh),
            (sh_s, osc), (sh_wa, owa), (sh_ha, oha), (sh_cx, ocx),
            (sh_cy, ocy))
    for sharr, ohbm in outs:
        pltpu.sync_copy(sharr.at[pl.ds(wbase, 384)], stage)
        pltpu.sync_copy(stage, ohbm.at[pl.ds(wbase, 384)])


def _sc_stage(dx, dy, dw, dh, s, wa, ha, cx, cy):
    mesh = plsc.VectorSubcoreMesh(core_axis_name="c", subcore_axis_name="s",
                                  num_cores=1, num_subcores=_NW)
    f32 = jnp.float32
    out_type = tuple(jax.ShapeDtypeStruct((_OUTN,), f32) for _ in range(9))
    vm_f = lambda: pltpu.VMEM((_CH,), f32)
    vm_i = lambda: pltpu.VMEM((_CH,), jnp.int32)
    scratch = [vm_f() for _ in range(9)] + [vm_i(), vm_i(),
               pltpu.VMEM((_CH // 128, 128), jnp.int32),
               pltpu.VMEM((48,), jnp.int32),
               pltpu.VMEM((768,), jnp.int32),
               pltpu.VMEM_SHARED((27 * 768,), jnp.int32)]
    scratch += [pltpu.VMEM_SHARED((_OUTN,), f32) for _ in range(9)]
    scratch += [pltpu.SemaphoreType.DMA]
    fn = pl.kernel(_sc_body, out_type=out_type, mesh=mesh,
                   scratch_types=tuple(scratch))
    return fn(dx, dy, dw, dh, s, wa, ha, cx, cy)


def _tc_body(dx_ref, dy_ref, dw_ref, dh_ref, s_ref, wa_ref, ha_ref, cx_ref,
             cy_ref, ob_ref, os_ref, valid_ref,
             x1_ref, y1_ref, x2_ref, y2_ref, ar_ref):
    dx = dx_ref[...]
    dy = dy_ref[...]
    dw = dw_ref[...]
    dh = dh_ref[...]
    s_in = s_ref[...]
    wa = wa_ref[...]
    ha = ha_ref[...]
    cxa = cx_ref[...]
    cya = cy_ref[...]

    lin = (lax.broadcasted_iota(jnp.int32, (_C_ROWS, _C_COLS), 0) * _C_COLS
           + lax.broadcasted_iota(jnp.int32, (_C_ROWS, _C_COLS), 1))
    col = lax.broadcasted_iota(jnp.int32, (1, _C_COLS), 1)
    real = lin < _PRE_NMS_N

    pred_ctr_x = dx * wa + cxa
    pred_ctr_y = dy * ha + cya
    pred_w = jnp.exp(dw) * wa
    pred_h = jnp.exp(dh) * ha
    x1 = jnp.clip(pred_ctr_x - 0.5 * pred_w, 0.0, _IMG_W - 1.0)
    y1 = jnp.clip(pred_ctr_y - 0.5 * pred_h, 0.0, _IMG_H - 1.0)
    x2 = jnp.clip(pred_ctr_x + 0.5 * pred_w, 0.0, _IMG_W - 1.0)
    y2 = jnp.clip(pred_ctr_y + 0.5 * pred_h, 0.0, _IMG_H - 1.0)
    ws = x2 - x1 + 1.0
    hs = y2 - y1 + 1.0
    size_ok = (ws >= _RPN_MIN_SIZE) & (hs >= _RPN_MIN_SIZE)
    s_eff = jnp.where(size_ok, s_in, jnp.float32(-1e9))
    s_eff = jnp.where(real, s_eff, -jnp.inf)
    areas = (x2 - x1) * (y2 - y1)
    x1_ref[...] = x1
    y1_ref[...] = y1
    x2_ref[...] = x2
    y2_ref[...] = y2
    ar_ref[...] = areas

    neg_inf = jnp.float32(-jnp.inf)
    big_i = jnp.int32(2**30)

    def select(valid):
        masked = jnp.where(valid, s_eff, neg_inf)
        m = jnp.max(masked)
        p = jnp.min(jnp.where(masked == m, lin, big_i))
        return m, p

    def extract(ref, r, c):
        row = ref[pl.ds(r, 1), :]
        return jnp.sum(jnp.where(col == c, row, 0.0))

    def emit_and_suppress(t, p, s_out, valid):
        r = p // _C_COLS
        c = p % _C_COLS
        x1i = extract(x1_ref, r, c)
        y1i = extract(y1_ref, r, c)
        x2i = extract(x2_ref, r, c)
        y2i = extract(y2_ref, r, c)
        ai = extract(ar_ref, r, c)
        ob_ref[t, 0] = x1i
        ob_ref[t, 1] = y1i
        ob_ref[t, 2] = x2i
        ob_ref[t, 3] = y2i
        os_ref[t] = s_out
        xx1 = jnp.maximum(x1i, x1)
        yy1 = jnp.maximum(y1i, y1)
        xx2 = jnp.minimum(x2i, x2)
        yy2 = jnp.minimum(y2i, y2)
        iw = jnp.maximum(xx2 - xx1, 0.0)
        ih = jnp.maximum(yy2 - yy1, 0.0)
        inter = iw * ih
        iou = inter / (ai + areas - inter + jnp.float32(1e-8))
        return valid & (iou <= _NMS_THRESH) & (lin != p)

    m0, p0 = select(real)
    valid_ref[...] = emit_and_suppress(0, p0, m0, real).astype(jnp.float32)

    def nms_step(t, carry):
        valid = valid_ref[...] != 0.0
        m, p = select(valid)
        empty = m == neg_inf
        p = jnp.where(empty, p0, p)
        s_out = jnp.where(empty, m0, m)
        valid_ref[...] = emit_and_suppress(t, p, s_out, valid).astype(jnp.float32)
        return carry

    lax.fori_loop(1, _POST_NMS_N, nms_step, jnp.int32(0))


def _tc_stage(arrs):
    vm = lambda: pltpu.VMEM((_C_ROWS, _C_COLS), jnp.float32)
    return pl.pallas_call(
        _tc_body,
        out_shape=(jax.ShapeDtypeStruct((_POST_NMS_N, 4), jnp.float32),
                   jax.ShapeDtypeStruct((_POST_NMS_N,), jnp.float32)),
        out_specs=(pl.BlockSpec(memory_space=pltpu.SMEM),
                   pl.BlockSpec(memory_space=pltpu.SMEM)),
        scratch_shapes=[vm(), vm(), vm(), vm(), vm(), vm()],
    )(*arrs)


def kernel(delta, score):
    wa, ha, cxa, cya = _anchor_consts_1d()
    d = jnp.reshape(delta, (-1, 4))
    padz = jnp.zeros((_NP - _N,), jnp.float32)
    cat = lambda c: jnp.concatenate([c, padz])
    dx = cat(d[:, 0])
    dy = cat(d[:, 1])
    dw = cat(d[:, 2])
    dh = cat(d[:, 3])
    s = jnp.concatenate([jnp.reshape(score[:, :, :, 9:], (-1,)),
                         jnp.full((_NP - _N,), -jnp.inf)])
    outs = _sc_stage(dx, dy, dw, dh, s,
                     jnp.asarray(wa), jnp.asarray(ha),
                     jnp.asarray(cxa), jnp.asarray(cya))
    comp = [jnp.reshape(o[:_C_ROWS * _C_COLS], (_C_ROWS, _C_COLS))
            for o in outs]
    return _tc_stage(comp)


# speedup vs baseline: 4.5958x; 1.0285x over previous
import functools

import numpy as np
import jax
import jax.numpy as jnp
from jax import lax
from jax.experimental import pallas as pl
from jax.experimental.pallas import tpu as pltpu
from jax.experimental.pallas import tpu_sc as plsc

_ANCHOR_SCALE = (8.0, 16.0, 32.0)
_ANCHOR_RATIO = (0.5, 1.0, 2.0)
_FEAT_STRIDE = 16
_RPN_MIN_SIZE = 16.0
_PRE_NMS_N = 6000
_POST_NMS_N = 300
_NMS_THRESH = 0.7
_IMG_W = 800.0
_IMG_H = 800.0
_MAP = 50
_N = _MAP * _MAP * 9
_NP = 22528
_NW = 16
_CH = _NP // _NW
_NVR = _CH // 16
_OUTN = 8192
_DUMP = 6144
_C_ROWS = 48
_C_COLS = 128


def _np_anchors():
    base_size = 16.0
    base = np.array([0.0, 0.0, base_size - 1.0, base_size - 1.0], dtype=np.float32)
    w = base[2] - base[0] + 1.0
    h = base[3] - base[1] + 1.0
    x_ctr = base[0] + 0.5 * (w - 1.0)
    y_ctr = base[1] + 0.5 * (h - 1.0)
    size = w * h
    anchors = []
    for r in _ANCHOR_RATIO:
        size_r = size / r
        ws = np.round(np.sqrt(size_r))
        hs = np.round(ws * r)
        for s in _ANCHOR_SCALE:
            wss = ws * s
            hss = hs * s
            anchors.append([x_ctr - 0.5 * (wss - 1.0), y_ctr - 0.5 * (hss - 1.0),
                            x_ctr + 0.5 * (wss - 1.0), y_ctr + 0.5 * (hss - 1.0)])
    anchors = np.array(anchors, dtype=np.float32)
    sx = np.arange(_MAP, dtype=np.float32) * _FEAT_STRIDE
    sy = np.arange(_MAP, dtype=np.float32) * _FEAT_STRIDE
    sxg, syg = np.meshgrid(sx, sy)
    shifts = np.stack([sxg.ravel(), syg.ravel(), sxg.ravel(), syg.ravel()], axis=1)
    all_anchors = (anchors[None, :, :] + shifts[:, None, :]).reshape(-1, 4)
    return all_anchors.astype(np.float32)


def _pad1d(col, fill):
    out = np.full((_NP,), fill, dtype=np.float32)
    out[:_N] = col
    return out


@functools.lru_cache(maxsize=1)
def _anchor_consts_1d():
    a = _np_anchors()
    widths = a[:, 2] - a[:, 0] + 1.0
    heights = a[:, 3] - a[:, 1] + 1.0
    ctr_x = a[:, 0] + 0.5 * widths
    ctr_y = a[:, 1] + 0.5 * heights
    return (_pad1d(widths, 16.0), _pad1d(heights, 16.0),
            _pad1d(ctr_x, 0.0), _pad1d(ctr_y, 0.0))


def _sc_sortable(f):
    m = lax.bitcast_convert_type(f, jnp.int32)
    neg = jnp.bitwise_xor(jnp.bitwise_not(m), jnp.int32(-2**31))
    return jnp.where(m >= 0, m, neg)


def _dgather16(v, idx):
    return lax.gather(
        v, idx.reshape(16, 1),
        dimension_numbers=lax.GatherDimensionNumbers(
            offset_dims=(), collapsed_slice_dims=(0,), start_index_map=(0,)),
        slice_sizes=(1,),
        mode=lax.GatherScatterMode.PROMISE_IN_BOUNDS)


def _sc_body(dxh, dyh, dwh, dhh, sh, wah, hah, cxh, cyh,
             odx, ody, odw, odh, osc, owa, oha, ocx, ocy,
             vdx, vdy, vdw, vdh, vs, vwa, vha, vcx, vcy,
             vv, vlin, vpos, cnt3buf, rdbuf, shared,
             sh_dx, sh_dy, sh_dw, sh_dh, sh_s, sh_wa, sh_ha, sh_cx, sh_cy,
             sem):
    sid = lax.axis_index("s")
    base = sid * _CH
    iota16 = lax.iota(jnp.int32, 16)
    K = jnp.int32(_PRE_NMS_N)

    def butterfly(v):
        for k in (1, 2, 4, 8):
            v = v + _dgather16(v, jnp.bitwise_xor(iota16, jnp.int32(k)))
        return v

    def prefix16(v):
        for k in (1, 2, 4, 8):
            sh_ = _dgather16(v, jnp.maximum(iota16 - k, 0))
            v = v + jnp.where(iota16 >= k, sh_, 0)
        return v

    def b2i(m):
        return jnp.where(m, jnp.int32(1), jnp.int32(0))

    in_cps = [pltpu.async_copy(href.at[pl.ds(base, _CH)], vref, sem)
              for href, vref in ((dxh, vdx), (dyh, vdy), (dwh, vdw),
                                 (dhh, vdh), (sh, vs), (wah, vwa),
                                 (hah, vha), (cxh, vcx), (cyh, vcy))]
    for cp in in_cps:
        cp.wait()

    def transform(j, carry):
        sl = pl.ds(j * 16, 16)
        dx = vdx[sl]
        dy = vdy[sl]
        dw = vdw[sl]
        dh = vdh[sl]
        wa = vwa[sl]
        ha = vha[sl]
        cxa = vcx[sl]
        cya = vcy[sl]
        s_in = vs[sl]
        lin = base + j * 16 + iota16
        pred_ctr_x = dx * wa + cxa
        pred_ctr_y = dy * ha + cya
        pred_w = jnp.exp(dw) * wa
        pred_h = jnp.exp(dh) * ha
        x1 = jnp.clip(pred_ctr_x - 0.5 * pred_w, 0.0, _IMG_W - 1.0)
        y1 = jnp.clip(pred_ctr_y - 0.5 * pred_h, 0.0, _IMG_H - 1.0)
        x2 = jnp.clip(pred_ctr_x + 0.5 * pred_w, 0.0, _IMG_W - 1.0)
        y2 = jnp.clip(pred_ctr_y + 0.5 * pred_h, 0.0, _IMG_H - 1.0)
        size_ok = ((x2 - x1 + 1.0 >= _RPN_MIN_SIZE)
                   & (y2 - y1 + 1.0 >= _RPN_MIN_SIZE))
        s_eff = jnp.where(size_ok, s_in, jnp.float32(-1e9))
        s_eff = jnp.where(lin < _N, s_eff, jnp.float32(-jnp.inf))
        vv[sl] = _sc_sortable(s_eff + 0.0)
        vlin[sl] = lin
        return carry
    lax.fori_loop(0, _NVR, transform, jnp.int32(0), unroll=4)

    def global_count3(step, c0, c1, c2):
        cnt3buf[pl.ds(0, 16)] = c0
        cnt3buf[pl.ds(16, 16)] = c1
        cnt3buf[pl.ds(32, 16)] = c2
        woff = pl.multiple_of(step * 768 + sid * 48, 16)
        pltpu.sync_copy(cnt3buf, shared.at[pl.ds(woff, 48)])
        plsc.subcore_barrier()
        roff = pl.multiple_of(step * 768, 16)
        pltpu.sync_copy(shared.at[pl.ds(roff, 768)], rdbuf)

        def red(w, accs):
            a0, a1, a2 = accs
            return (a0 + rdbuf[pl.ds(w * 48, 16)],
                    a1 + rdbuf[pl.ds(w * 48 + 16, 16)],
                    a2 + rdbuf[pl.ds(w * 48 + 32, 16)])
        z = jnp.zeros((16,), jnp.int32)
        a0, a1, a2 = lax.fori_loop(0, _NW, red, (z, z, z), unroll=4)
        return butterfly(a0), butterfly(a1), butterfly(a2)

    def lanes_gt3(t0, t1, t2):
        def body(j, accs):
            a0, a1, a2 = accs
            v = vv[pl.ds(j * 16, 16)]
            return (a0 + b2i(v > t0), a1 + b2i(v > t1), a2 + b2i(v > t2))
        z = jnp.zeros((16,), jnp.int32)
        return lax.fori_loop(0, _NVR, body, (z, z, z), unroll=8)

    int_min = jnp.full((16,), -2**31, jnp.int32)
    neg1 = jnp.full((16,), -1, jnp.int32)
    c0, _, _ = lanes_gt3(neg1, neg1, neg1)
    tot_m1, _, _ = global_count3(jnp.int32(0), c0, c0, c0)
    l = jnp.where(tot_m1 >= K, neg1, int_min)

    def vstep(t, l):
        hi = jnp.int32(30) - 2 * t
        lo = hi - 1
        ca = l + (jnp.int32(1) << hi)
        cb = l + (jnp.int32(1) << lo)
        cc = ca + (jnp.int32(1) << lo)
        n_a, n_b, n_c = lanes_gt3(ca, cb, cc)
        ta, tb, tc = global_count3(t + 1, n_a, n_b, n_c)
        return jnp.where(tc >= K, cc,
                         jnp.where(ta >= K, ca,
                                   jnp.where(tb >= K, cb, l)))
    l = lax.fori_loop(0, 15, vstep, l)
    cz = l + jnp.int32(1)
    n_z0, _, _ = lanes_gt3(cz, cz, cz)
    tz, _, _ = global_count3(jnp.int32(16), n_z0, n_z0, n_z0)
    l = jnp.where(tz >= K, cz, l)
    a_thr = l + jnp.int32(1)

    def lanes_gt_eq(a):
        def body(j, accs):
            a0, a1 = accs
            v = vv[pl.ds(j * 16, 16)]
            return (a0 + b2i(v > a), a1 + b2i(v == a))
        z = jnp.zeros((16,), jnp.int32)
        return lax.fori_loop(0, _NVR, body, (z, z), unroll=8)
    g_c, e_c = lanes_gt_eq(a_thr)
    n_gt, _, _ = global_count3(jnp.int32(17), g_c, e_c, e_c)
    need = K - n_gt

    def lanes_eq_lt3(c0_, c1_, c2_):
        def body(j, accs):
            a0, a1, a2 = accs
            sl = pl.ds(j * 16, 16)
            eq = vv[sl] == a_thr
            lin = vlin[sl]
            return (a0 + b2i(eq & (lin < c0_)),
                    a1 + b2i(eq & (lin < c1_)),
                    a2 + b2i(eq & (lin < c2_)))
        z = jnp.zeros((16,), jnp.int32)
        return lax.fori_loop(0, _NVR, body, (z, z, z), unroll=8)

    def istep(t, li):
        hi = jnp.int32(14) - 2 * t
        lo = hi - 1
        ca = li + (jnp.int32(1) << hi)
        cb = li + (jnp.int32(1) << lo)
        cc = ca + (jnp.int32(1) << lo)
        n_a, n_b, n_c = lanes_eq_lt3(ca, cb, cc)
        ta, tb, tc = global_count3(t + 18, n_a, n_b, n_c)
        return jnp.where(tc < need, cc,
                         jnp.where(ta < need, ca,
                                   jnp.where(tb < need, cb, li)))
    li = lax.fori_loop(0, 7, istep, jnp.zeros((16,), jnp.int32))
    ciz = li + jnp.int32(1)
    n_i0, _, _ = lanes_eq_lt3(ciz, ciz, ciz)
    tiz, _, _ = global_count3(jnp.int32(25), n_i0, n_i0, n_i0)
    li = jnp.where(tiz < need, ciz, li)
    cfin = li + jnp.int32(1)

    def lanes_member(j):
        sl = pl.ds(j * 16, 16)
        v = vv[sl]
        return (v > a_thr) | ((v == a_thr) & (vlin[sl] < cfin))

    def mcount(j, acc):
        return acc + b2i(lanes_member(j))
    mvec = lax.fori_loop(0, _NVR, mcount, jnp.zeros((16,), jnp.int32),
                         unroll=8)
    cnt3buf[pl.ds(0, 16)] = mvec
    cnt3buf[pl.ds(16, 16)] = mvec
    cnt3buf[pl.ds(32, 16)] = mvec
    woff = pl.multiple_of(26 * 768 + sid * 48, 16)
    pltpu.sync_copy(cnt3buf, shared.at[pl.ds(woff, 48)])
    plsc.subcore_barrier()
    roff = pl.multiple_of(26 * 768, 16)
    pltpu.sync_copy(shared.at[pl.ds(roff, 768)], rdbuf)

    def offred(w, acc):
        row = rdbuf[pl.ds(w * 48, 16)]
        return acc + jnp.where(w < sid, row, 0)
    offacc = lax.fori_loop(0, _NW, offred, jnp.zeros((16,), jnp.int32),
                           unroll=4)
    off_w = butterfly(offacc)

    def posloop(j, running):
        sl = pl.ds(j * 16, 16)
        member = lanes_member(j)
        mi = b2i(member)
        incl = prefix16(mi)
        lin = vlin[sl]
        dump = _DUMP + jnp.bitwise_and(lin, jnp.int32(2047))
        pos = jnp.where(member, off_w + running + incl - 1, dump)
        r2 = j // 8
        c2 = (j % 8) * 16
        vpos[r2, pl.ds(c2, 16)] = pos
        return running + butterfly(mi)
    lax.fori_loop(0, _NVR, posloop, jnp.zeros((16,), jnp.int32), unroll=4)

    pairs = ((vdx, sh_dx), (vdy, sh_dy), (vdw, sh_dw), (vdh, sh_dh),
             (vs, sh_s), (vwa, sh_wa), (vha, sh_ha), (vcx, sh_cx),
             (vcy, sh_cy))
    chunks = _CH // 128
    for j in range(chunks):
        sl = pl.ds(j * 128, 128)
        row = vpos.at[j]
        cps = [pltpu.async_copy(src.at[sl], dst.at[row], sem)
               for src, dst in pairs]
        for cp in cps:
            cp.wait()
    plsc.subcore_barrier()
    wbase = pl.multiple_of(sid * 384, 16)
    stage = vdx.at[pl.ds(0, 384)]
    outs = ((sh_dx, odx), (sh_dy, ody), (sh_dw, odw), (sh_dh, odh),
            (sh_s, osc), (sh_wa, owa), (sh_ha, oha), (sh_cx, ocx),
            (sh_cy, ocy))
    for sharr, ohbm in outs:
        pltpu.sync_copy(sharr.at[pl.ds(wbase, 384)], stage)
        pltpu.sync_copy(stage, ohbm.at[pl.ds(wbase, 384)])


def _sc_stage(dx, dy, dw, dh, s, wa, ha, cx, cy):
    mesh = plsc.VectorSubcoreMesh(core_axis_name="c", subcore_axis_name="s",
                                  num_cores=1, num_subcores=_NW)
    f32 = jnp.float32
    out_type = tuple(jax.ShapeDtypeStruct((_OUTN,), f32) for _ in range(9))
    vm_f = lambda: pltpu.VMEM((_CH,), f32)
    vm_i = lambda: pltpu.VMEM((_CH,), jnp.int32)
    scratch = [vm_f() for _ in range(9)] + [vm_i(), vm_i(),
               pltpu.VMEM((_CH // 128, 128), jnp.int32),
               pltpu.VMEM((48,), jnp.int32),
               pltpu.VMEM((768,), jnp.int32),
               pltpu.VMEM_SHARED((27 * 768,), jnp.int32)]
    scratch += [pltpu.VMEM_SHARED((_OUTN,), f32) for _ in range(9)]
    scratch += [pltpu.SemaphoreType.DMA]
    fn = pl.kernel(_sc_body, out_type=out_type, mesh=mesh,
                   scratch_types=tuple(scratch))
    return fn(dx, dy, dw, dh, s, wa, ha, cx, cy)


def _tc_body(dx_ref, dy_ref, dw_ref, dh_ref, s_ref, wa_ref, ha_ref, cx_ref,
             cy_ref, ob_ref, os_ref, coord_ref):
    dx = dx_ref[...]
    dy = dy_ref[...]
    dw = dw_ref[...]
    dh = dh_ref[...]
    s_in = s_ref[...]
    wa = wa_ref[...]
    ha = ha_ref[...]
    cxa = cx_ref[...]
    cya = cy_ref[...]

    lin = (lax.broadcasted_iota(jnp.int32, (_C_ROWS, _C_COLS), 0) * _C_COLS
           + lax.broadcasted_iota(jnp.int32, (_C_ROWS, _C_COLS), 1))
    col = lax.broadcasted_iota(jnp.int32, (1, _C_COLS), 1)
    real = lin < _PRE_NMS_N

    pred_ctr_x = dx * wa + cxa
    pred_ctr_y = dy * ha + cya
    pred_w = jnp.exp(dw) * wa
    pred_h = jnp.exp(dh) * ha
    x1 = jnp.clip(pred_ctr_x - 0.5 * pred_w, 0.0, _IMG_W - 1.0)
    y1 = jnp.clip(pred_ctr_y - 0.5 * pred_h, 0.0, _IMG_H - 1.0)
    x2 = jnp.clip(pred_ctr_x + 0.5 * pred_w, 0.0, _IMG_W - 1.0)
    y2 = jnp.clip(pred_ctr_y + 0.5 * pred_h, 0.0, _IMG_H - 1.0)
    ws = x2 - x1 + 1.0
    hs = y2 - y1 + 1.0
    size_ok = (ws >= _RPN_MIN_SIZE) & (hs >= _RPN_MIN_SIZE)
    s_eff = jnp.where(size_ok, s_in, jnp.float32(-1e9))
    s_eff = jnp.where(real, s_eff, -jnp.inf)
    areas = (x2 - x1) * (y2 - y1)
    coord_ref[:, pl.ds(0, 1), :] = x1.reshape(_C_ROWS, 1, _C_COLS)
    coord_ref[:, pl.ds(1, 1), :] = y1.reshape(_C_ROWS, 1, _C_COLS)
    coord_ref[:, pl.ds(2, 1), :] = x2.reshape(_C_ROWS, 1, _C_COLS)
    coord_ref[:, pl.ds(3, 1), :] = y2.reshape(_C_ROWS, 1, _C_COLS)
    coord_ref[:, pl.ds(4, 1), :] = areas.reshape(_C_ROWS, 1, _C_COLS)
    coord_ref[:, pl.ds(5, 1), :] = jnp.zeros((_C_ROWS, 1, _C_COLS),
                                             jnp.float32)
    coord_ref[:, pl.ds(6, 1), :] = jnp.zeros((_C_ROWS, 1, _C_COLS),
                                             jnp.float32)
    coord_ref[:, pl.ds(7, 1), :] = jnp.zeros((_C_ROWS, 1, _C_COLS),
                                             jnp.float32)

    neg_inf = jnp.float32(-jnp.inf)
    big_i = jnp.int32(2**30)

    def select(masked):
        m = jnp.max(masked)
        p = jnp.min(jnp.where(masked == m, lin, big_i))
        return m, p

    def emit_and_suppress(t, p, s_out, masked):
        r = p // _C_COLS
        c = p % _C_COLS
        block = coord_ref[pl.ds(r, 1), :, :].reshape(8, _C_COLS)
        sel = jnp.sum(jnp.where(col == c, block, 0.0), axis=1)
        x1i = sel[0]
        y1i = sel[1]
        x2i = sel[2]
        y2i = sel[3]
        ai = sel[4]
        ob_ref[t, 0] = x1i
        ob_ref[t, 1] = y1i
        ob_ref[t, 2] = x2i
        ob_ref[t, 3] = y2i
        os_ref[t] = s_out
        xx1 = jnp.maximum(x1i, x1)
        yy1 = jnp.maximum(y1i, y1)
        xx2 = jnp.minimum(x2i, x2)
        yy2 = jnp.minimum(y2i, y2)
        iw = jnp.maximum(xx2 - xx1, 0.0)
        ih = jnp.maximum(yy2 - yy1, 0.0)
        inter = iw * ih
        iou = inter / (ai + areas - inter + jnp.float32(1e-8))
        keep = (iou <= _NMS_THRESH) & (lin != p)
        return jnp.where(keep, masked, neg_inf)

    masked0 = jnp.where(real, s_eff, neg_inf)
    m0, p0 = select(masked0)
    masked = emit_and_suppress(0, p0, m0, masked0)

    def nms_step(t, masked):
        m, p = select(masked)
        empty = m == neg_inf
        p = jnp.where(empty, p0, p)
        s_out = jnp.where(empty, m0, m)
        return emit_and_suppress(t, p, s_out, masked)

    lax.fori_loop(1, _POST_NMS_N, nms_step, masked)


def _tc_stage(arrs):
    return pl.pallas_call(
        _tc_body,
        out_shape=(jax.ShapeDtypeStruct((_POST_NMS_N, 4), jnp.float32),
                   jax.ShapeDtypeStruct((_POST_NMS_N,), jnp.float32)),
        out_specs=(pl.BlockSpec(memory_space=pltpu.SMEM),
                   pl.BlockSpec(memory_space=pltpu.SMEM)),
        scratch_shapes=[pltpu.VMEM((_C_ROWS, 8, _C_COLS), jnp.float32)],
    )(*arrs)


def kernel(delta, score):
    wa, ha, cxa, cya = _anchor_consts_1d()
    d = jnp.reshape(delta, (-1, 4))
    padz = jnp.zeros((_NP - _N,), jnp.float32)
    cat = lambda c: jnp.concatenate([c, padz])
    dx = cat(d[:, 0])
    dy = cat(d[:, 1])
    dw = cat(d[:, 2])
    dh = cat(d[:, 3])
    s = jnp.concatenate([jnp.reshape(score[:, :, :, 9:], (-1,)),
                         jnp.full((_NP - _N,), -jnp.inf)])
    outs = _sc_stage(dx, dy, dw, dh, s,
                     jnp.asarray(wa), jnp.asarray(ha),
                     jnp.asarray(cxa), jnp.asarray(cya))
    comp = [jnp.reshape(o[:_C_ROWS * _C_COLS], (_C_ROWS, _C_COLS))
            for o in outs]
    return _tc_stage(comp)
